# speedup vs baseline: 2.1572x; 2.1572x over previous
import jax
import jax.numpy as jnp
from jax.experimental import pallas as pl

E, N, D, H, O, TOPK = 8, 2048, 1024, 2048, 1024, 2
NB = 8
BN = N // NB


def _gate_hidden_kernel(f_ref, g1_ref, h_ref):
    e = pl.program_id(0)
    nb = pl.program_id(1)
    rows = pl.ds(nb * BN, BN)
    prod = jnp.dot(f_ref[0], g1_ref[0], preferred_element_type=jnp.float32)

    @pl.when(e == 0)
    def _():
        h_ref[rows, :] = prod

    @pl.when(e > 0)
    def _():
        h_ref[rows, :] = h_ref[rows, :] + prod


def _routing_kernel(hpre_ref, gb1_ref, gw2_ref, gb2_ref, recept_ref,
                    w_ref, ranks_ref, totals_ref):
    h = jnp.maximum(hpre_ref[...] + gb1_ref[...], 0.0)
    logits = jnp.dot(h, gw2_ref[...], preferred_element_type=jnp.float32)
    logits = logits + gb2_ref[...]
    m = jnp.max(logits, axis=1, keepdims=True)
    ex = jnp.exp(logits - m)
    probs = ex / jnp.sum(ex, axis=1, keepdims=True)
    scores = probs + recept_ref[...]

    lane = jax.lax.broadcasted_iota(jnp.int32, scores.shape, 1)
    t1 = jnp.max(scores, axis=1, keepdims=True)
    i1 = jnp.min(jnp.where(scores == t1, lane, E), axis=1, keepdims=True)
    masked = jnp.where(lane == i1, -jnp.inf, scores)
    t2 = jnp.max(masked, axis=1, keepdims=True)
    i2 = jnp.min(jnp.where(masked == t2, lane, E), axis=1, keepdims=True)

    denom = t1 + t2
    w_ref[...] = (jnp.where(lane == i1, t1 / denom, 0.0)
                  + jnp.where(lane == i2, t2 / denom, 0.0))
    ranks = jnp.where(lane == i1, 0, jnp.where(lane == i2, 1, 2)).astype(jnp.int32)
    ranks_ref[...] = ranks
    totals_ref[...] = jnp.sum(ranks, axis=0, keepdims=True)


def _expert_ffn_kernel(f_ref, w1_ref, b1_ref, w2_ref, b2_ref, wmat_ref, out_ref):
    e = pl.program_id(0)
    nb = pl.program_id(1)
    rows = pl.ds(nb * BN, BN)
    x = f_ref[0]
    hid = jnp.maximum(jnp.dot(x, w1_ref[0], preferred_element_type=jnp.float32)
                      + b1_ref[0], 0.0)
    y = jnp.dot(hid, w2_ref[0], preferred_element_type=jnp.float32) + b2_ref[0]
    wl = wmat_ref[rows, :]
    lane = jax.lax.broadcasted_iota(jnp.int32, wl.shape, 1)
    wcol = jnp.sum(jnp.where(lane == e, wl, 0.0), axis=1, keepdims=True)
    contrib = y * wcol

    @pl.when(e == 0)
    def _():
        out_ref[rows, :] = contrib

    @pl.when(e > 0)
    def _():
        out_ref[rows, :] = out_ref[rows, :] + contrib


def kernel(features_list, receptivity, expert_w1, expert_b1, expert_w2,
           expert_b2, gate_w1, gate_b1, gate_w2, gate_b2):
    g1 = gate_w1.reshape(E, D, H)
    receptT = jnp.transpose(receptivity, (1, 0, 2))[:, :, 0]
    gb1r = gate_b1.reshape(1, H)
    gb2r = gate_b2.reshape(1, E)
    b1r = expert_b1.reshape(E, 1, H)
    b2r = expert_b2.reshape(E, 1, O)

    h_pre = pl.pallas_call(
        _gate_hidden_kernel,
        grid=(E, NB),
        in_specs=[
            pl.BlockSpec((1, BN, D), lambda e, nb: (e, nb, 0)),
            pl.BlockSpec((1, D, H), lambda e, nb: (e, 0, 0)),
        ],
        out_specs=pl.BlockSpec((N, H), lambda e, nb: (0, 0)),
        out_shape=jax.ShapeDtypeStruct((N, H), jnp.float32),
    )(features_list, g1)

    wmat, ranksT, totals_row = pl.pallas_call(
        _routing_kernel,
        grid=(1,),
        in_specs=[
            pl.BlockSpec((N, H), lambda i: (0, 0)),
            pl.BlockSpec((1, H), lambda i: (0, 0)),
            pl.BlockSpec((H, E), lambda i: (0, 0)),
            pl.BlockSpec((1, E), lambda i: (0, 0)),
            pl.BlockSpec((N, E), lambda i: (0, 0)),
        ],
        out_specs=[
            pl.BlockSpec((N, E), lambda i: (0, 0)),
            pl.BlockSpec((N, E), lambda i: (0, 0)),
            pl.BlockSpec((1, E), lambda i: (0, 0)),
        ],
        out_shape=[
            jax.ShapeDtypeStruct((N, E), jnp.float32),
            jax.ShapeDtypeStruct((N, E), jnp.int32),
            jax.ShapeDtypeStruct((1, E), jnp.int32),
        ],
    )(h_pre, gb1r, gate_w2, gb2r, receptT)

    final_out = pl.pallas_call(
        _expert_ffn_kernel,
        grid=(E, NB),
        in_specs=[
            pl.BlockSpec((1, BN, D), lambda e, nb: (e, nb, 0)),
            pl.BlockSpec((1, D, H), lambda e, nb: (e, 0, 0)),
            pl.BlockSpec((1, 1, H), lambda e, nb: (e, 0, 0)),
            pl.BlockSpec((1, H, O), lambda e, nb: (e, 0, 0)),
            pl.BlockSpec((1, 1, O), lambda e, nb: (e, 0, 0)),
            pl.BlockSpec((N, E), lambda e, nb: (0, 0)),
        ],
        out_specs=pl.BlockSpec((N, O), lambda e, nb: (0, 0)),
        out_shape=jax.ShapeDtypeStruct((N, O), jnp.float32),
    )(features_list, expert_w1, b1r, expert_w2, b2r, wmat)

    ranks = jnp.transpose(ranksT)
    totals = totals_row.reshape(E)
    return final_out, ranks, totals


# speedup vs baseline: 2.1625x; 1.0025x over previous
import jax
import jax.numpy as jnp
from jax.experimental import pallas as pl

E, N, D, H, O, TOPK = 8, 2048, 1024, 2048, 1024, 2
NB = 8
BN = N // NB


def _gate_hidden_kernel(f_ref, g1_ref, h_ref):
    e = pl.program_id(0)
    nb = pl.program_id(1)
    rows = pl.ds(nb * BN, BN)
    prod = jnp.dot(f_ref[0], g1_ref[0], preferred_element_type=jnp.float32)

    @pl.when(e == 0)
    def _():
        h_ref[rows, :] = prod

    @pl.when(e > 0)
    def _():
        h_ref[rows, :] = h_ref[rows, :] + prod


def _routing_kernel(hpre_ref, gb1_ref, gw2_ref, gb2_ref, recept_ref,
                    w_ref, ranks_ref, totals_ref):
    h = jnp.maximum(hpre_ref[...] + gb1_ref[...], 0.0)
    logits = jnp.dot(h, gw2_ref[...], preferred_element_type=jnp.float32)
    logits = logits + gb2_ref[...]
    m = jnp.max(logits, axis=1, keepdims=True)
    ex = jnp.exp(logits - m)
    probs = ex / jnp.sum(ex, axis=1, keepdims=True)
    scores = probs + recept_ref[...]

    lane = jax.lax.broadcasted_iota(jnp.int32, scores.shape, 1)
    t1 = jnp.max(scores, axis=1, keepdims=True)
    i1 = jnp.min(jnp.where(scores == t1, lane, E), axis=1, keepdims=True)
    masked = jnp.where(lane == i1, -jnp.inf, scores)
    t2 = jnp.max(masked, axis=1, keepdims=True)
    i2 = jnp.min(jnp.where(masked == t2, lane, E), axis=1, keepdims=True)

    denom = t1 + t2
    w_ref[...] = (jnp.where(lane == i1, t1 / denom, 0.0)
                  + jnp.where(lane == i2, t2 / denom, 0.0))
    ranks = jnp.where(lane == i1, 0, jnp.where(lane == i2, 1, 2)).astype(jnp.int32)
    ranks_ref[...] = ranks
    totals_ref[...] = jnp.sum(ranks, axis=0, keepdims=True)


def _expert_ffn_kernel(f_ref, w1_ref, b1_ref, w2_ref, b2_ref, wmat_ref, out_ref):
    e = pl.program_id(0)
    nb = pl.program_id(1)
    rows = pl.ds(nb * BN, BN)
    x = f_ref[0].astype(jnp.bfloat16)
    hid = jnp.maximum(
        jnp.dot(x, w1_ref[0].astype(jnp.bfloat16),
                preferred_element_type=jnp.float32) + b1_ref[0], 0.0)
    y = jnp.dot(hid.astype(jnp.bfloat16), w2_ref[0].astype(jnp.bfloat16),
                preferred_element_type=jnp.float32) + b2_ref[0]
    wl = wmat_ref[rows, :]
    lane = jax.lax.broadcasted_iota(jnp.int32, wl.shape, 1)
    wcol = jnp.sum(jnp.where(lane == e, wl, 0.0), axis=1, keepdims=True)
    contrib = y * wcol

    @pl.when(e == 0)
    def _():
        out_ref[rows, :] = contrib

    @pl.when(e > 0)
    def _():
        out_ref[rows, :] = out_ref[rows, :] + contrib


def kernel(features_list, receptivity, expert_w1, expert_b1, expert_w2,
           expert_b2, gate_w1, gate_b1, gate_w2, gate_b2):
    g1 = gate_w1.reshape(E, D, H)
    receptT = jnp.transpose(receptivity, (1, 0, 2))[:, :, 0]
    gb1r = gate_b1.reshape(1, H)
    gb2r = gate_b2.reshape(1, E)
    b1r = expert_b1.reshape(E, 1, H)
    b2r = expert_b2.reshape(E, 1, O)

    h_pre = pl.pallas_call(
        _gate_hidden_kernel,
        grid=(E, NB),
        in_specs=[
            pl.BlockSpec((1, BN, D), lambda e, nb: (e, nb, 0)),
            pl.BlockSpec((1, D, H), lambda e, nb: (e, 0, 0)),
        ],
        out_specs=pl.BlockSpec((N, H), lambda e, nb: (0, 0)),
        out_shape=jax.ShapeDtypeStruct((N, H), jnp.float32),
    )(features_list, g1)

    wmat, ranksT, totals_row = pl.pallas_call(
        _routing_kernel,
        grid=(1,),
        in_specs=[
            pl.BlockSpec((N, H), lambda i: (0, 0)),
            pl.BlockSpec((1, H), lambda i: (0, 0)),
            pl.BlockSpec((H, E), lambda i: (0, 0)),
            pl.BlockSpec((1, E), lambda i: (0, 0)),
            pl.BlockSpec((N, E), lambda i: (0, 0)),
        ],
        out_specs=[
            pl.BlockSpec((N, E), lambda i: (0, 0)),
            pl.BlockSpec((N, E), lambda i: (0, 0)),
            pl.BlockSpec((1, E), lambda i: (0, 0)),
        ],
        out_shape=[
            jax.ShapeDtypeStruct((N, E), jnp.float32),
            jax.ShapeDtypeStruct((N, E), jnp.int32),
            jax.ShapeDtypeStruct((1, E), jnp.int32),
        ],
    )(h_pre, gb1r, gate_w2, gb2r, receptT)

    final_out = pl.pallas_call(
        _expert_ffn_kernel,
        grid=(E, NB),
        in_specs=[
            pl.BlockSpec((1, BN, D), lambda e, nb: (e, nb, 0)),
            pl.BlockSpec((1, D, H), lambda e, nb: (e, 0, 0)),
            pl.BlockSpec((1, 1, H), lambda e, nb: (e, 0, 0)),
            pl.BlockSpec((1, H, O), lambda e, nb: (e, 0, 0)),
            pl.BlockSpec((1, 1, O), lambda e, nb: (e, 0, 0)),
            pl.BlockSpec((N, E), lambda e, nb: (0, 0)),
        ],
        out_specs=pl.BlockSpec((N, O), lambda e, nb: (0, 0)),
        out_shape=jax.ShapeDtypeStruct((N, O), jnp.float32),
    )(features_list, expert_w1, b1r, expert_w2, b2r, wmat)

    ranks = jnp.transpose(ranksT)
    totals = totals_row.reshape(E)
    return final_out, ranks, totals


# speedup vs baseline: 2.2057x; 1.0200x over previous
import jax
import jax.numpy as jnp
from jax import lax
from jax.experimental import pallas as pl
from jax.experimental.pallas import tpu as pltpu
from jax.experimental.pallas import tpu_sc as plsc

E, N, D, H, O, TOPK = 8, 2048, 1024, 2048, 1024, 2
KN = N * TOPK
NB = 8
BN = N // NB
BP = 256
P = KN + E * BP
PB = P // BP
NC, NS = 2, 16
NW = NC * NS
APT = KN // NW


def _gate_hidden_kernel(f_ref, g1_ref, h_ref):
    e = pl.program_id(0)
    nb = pl.program_id(1)
    rows = pl.ds(nb * BN, BN)
    prod = jnp.dot(f_ref[0], g1_ref[0], preferred_element_type=jnp.float32)

    @pl.when(e == 0)
    def _():
        h_ref[rows, :] = prod

    @pl.when(e > 0)
    def _():
        h_ref[rows, :] = h_ref[rows, :] + prod


def _routing_kernel(hpre_ref, gb1_ref, gw2_ref, gb2_ref, recept_ref,
                    wk_ref, ranks_ref, totals_ref, pos_ref, src_ref, be_ref):
    h = jnp.maximum(hpre_ref[...] + gb1_ref[...], 0.0)
    logits = jnp.dot(h, gw2_ref[...], preferred_element_type=jnp.float32)
    logits = logits + gb2_ref[...]
    m = jnp.max(logits, axis=1, keepdims=True)
    ex = jnp.exp(logits - m)
    probs = ex / jnp.sum(ex, axis=1, keepdims=True)
    scores = probs + recept_ref[...]

    lane = lax.broadcasted_iota(jnp.int32, scores.shape, 1)
    t1 = jnp.max(scores, axis=1, keepdims=True)
    i1 = jnp.min(jnp.where(scores == t1, lane, E), axis=1, keepdims=True)
    masked = jnp.where(lane == i1, -jnp.inf, scores)
    t2 = jnp.max(masked, axis=1, keepdims=True)
    i2 = jnp.min(jnp.where(masked == t2, lane, E), axis=1, keepdims=True)

    denom = t1 + t2
    wk_ref[...] = jnp.concatenate([t1 / denom, t2 / denom], axis=1)
    ranks = jnp.where(lane == i1, 0, jnp.where(lane == i2, 1, 2)).astype(jnp.int32)
    ranks_ref[...] = ranks
    totals_ref[...] = jnp.sum(ranks, axis=0, keepdims=True)

    lane16 = lax.broadcasted_iota(jnp.int32, (N, 2 * E), 1)
    ek = lane16 % E
    kk = lane16 // E
    sel = jnp.where(kk == 0, i1, i2)
    onehot = (sel == ek).astype(jnp.float32)
    tri = (lax.broadcasted_iota(jnp.int32, (N, N), 1)
           < lax.broadcasted_iota(jnp.int32, (N, N), 0)).astype(jnp.float32)
    cum = jnp.dot(tri, onehot, preferred_element_type=jnp.float32)
    tot = jnp.sum(onehot, axis=0, keepdims=True)
    tot0 = tot[:, 0:E]
    counts = tot0 + tot[:, E:2 * E]
    pc = jnp.ceil(counts / BP) * BP
    u8 = (lax.broadcasted_iota(jnp.int32, (E, E), 0)
          < lax.broadcasted_iota(jnp.int32, (E, E), 1)).astype(jnp.float32)
    starts = jnp.dot(jnp.broadcast_to(pc, (E, E)), u8,
                     preferred_element_type=jnp.float32)[0:1, :]
    cum0 = cum[:, 0:E]
    cum1 = cum[:, E:2 * E] + tot0
    oh0 = onehot[:, 0:E]
    oh1 = onehot[:, E:2 * E]
    pos0 = jnp.sum(oh0 * (starts + cum0), axis=1, keepdims=True)
    pos1 = jnp.sum(oh1 * (starts + cum1), axis=1, keepdims=True)
    pos_ref[...] = jnp.concatenate([pos0, pos1], axis=1).astype(jnp.int32)
    tok = lax.broadcasted_iota(jnp.int32, (N, 1), 0)
    src_ref[...] = jnp.concatenate([i1 * N + tok, i2 * N + tok], axis=1)
    bst = (lax.broadcasted_iota(jnp.int32, (PB, E), 0) * BP).astype(jnp.float32)
    startsb = jnp.broadcast_to(starts, (PB, E))
    be_ref[...] = (jnp.sum((startsb <= bst).astype(jnp.int32), axis=1,
                           keepdims=True) - 1)


def _dispatch_body(feat_ref, src_ref, pos_ref, x_ref, sidx_v, pidx_v, rows_v, sem):
    wid = lax.axis_index("s") * NC + lax.axis_index("c")
    base = wid * APT
    for j in range(APT // 32):
        off = base + j * 32
        pltpu.sync_copy(src_ref.at[pl.ds(off, 32)], sidx_v)
        pltpu.sync_copy(pos_ref.at[pl.ds(off, 32)], pidx_v)
        pltpu.async_copy(feat_ref.at[sidx_v], rows_v, sem).wait()
        pltpu.async_copy(rows_v, x_ref.at[pidx_v], sem).wait()


def _gather_back_body(y_ref, pos_ref, z_ref, pidx_v, rows_v, sem):
    wid = lax.axis_index("s") * NC + lax.axis_index("c")
    base = wid * APT
    for j in range(APT // 64):
        off = base + j * 64
        pltpu.sync_copy(pos_ref.at[pl.ds(off, 64)], pidx_v)
        pltpu.async_copy(y_ref.at[pidx_v], rows_v, sem).wait()
        pltpu.sync_copy(rows_v, z_ref.at[pl.ds(off, 64)])


def _grouped_ffn_kernel(be_ref, x_ref, w1_ref, b1_ref, w2_ref, b2_ref, y_ref):
    x = x_ref[...]
    hid = jnp.maximum(jnp.dot(x, w1_ref[0], preferred_element_type=jnp.float32)
                      + b1_ref[0], 0.0)
    y_ref[...] = jnp.dot(hid, w2_ref[0], preferred_element_type=jnp.float32) + b2_ref[0]


def _combine_kernel(z_ref, wk_ref, out_ref):
    z = z_ref[...]
    w0 = wk_ref[:, 0:1]
    w1 = wk_ref[:, 1:2]
    out_ref[...] = z[:, :O] * w0 + z[:, O:] * w1


def kernel(features_list, receptivity, expert_w1, expert_b1, expert_w2,
           expert_b2, gate_w1, gate_b1, gate_w2, gate_b2):
    g1 = gate_w1.reshape(E, D, H)
    receptT = jnp.transpose(receptivity, (1, 0, 2))[:, :, 0]
    gb1r = gate_b1.reshape(1, H)
    gb2r = gate_b2.reshape(1, E)
    b1r = expert_b1.reshape(E, 1, H)
    b2r = expert_b2.reshape(E, 1, O)
    feat_flat = features_list.reshape(E * N, D)

    h_pre = pl.pallas_call(
        _gate_hidden_kernel,
        grid=(E, NB),
        in_specs=[
            pl.BlockSpec((1, BN, D), lambda e, nb: (e, nb, 0)),
            pl.BlockSpec((1, D, H), lambda e, nb: (e, 0, 0)),
        ],
        out_specs=pl.BlockSpec((N, H), lambda e, nb: (0, 0)),
        out_shape=jax.ShapeDtypeStruct((N, H), jnp.float32),
    )(features_list, g1)

    wk, ranksT, totals_row, pos, src, be2d = pl.pallas_call(
        _routing_kernel,
        grid=(1,),
        in_specs=[
            pl.BlockSpec((N, H), lambda i: (0, 0)),
            pl.BlockSpec((1, H), lambda i: (0, 0)),
            pl.BlockSpec((H, E), lambda i: (0, 0)),
            pl.BlockSpec((1, E), lambda i: (0, 0)),
            pl.BlockSpec((N, E), lambda i: (0, 0)),
        ],
        out_specs=[
            pl.BlockSpec((N, TOPK), lambda i: (0, 0)),
            pl.BlockSpec((N, E), lambda i: (0, 0)),
            pl.BlockSpec((1, E), lambda i: (0, 0)),
            pl.BlockSpec((N, TOPK), lambda i: (0, 0)),
            pl.BlockSpec((N, TOPK), lambda i: (0, 0)),
            pl.BlockSpec((PB, 1), lambda i: (0, 0)),
        ],
        out_shape=[
            jax.ShapeDtypeStruct((N, TOPK), jnp.float32),
            jax.ShapeDtypeStruct((N, E), jnp.int32),
            jax.ShapeDtypeStruct((1, E), jnp.int32),
            jax.ShapeDtypeStruct((N, TOPK), jnp.int32),
            jax.ShapeDtypeStruct((N, TOPK), jnp.int32),
            jax.ShapeDtypeStruct((PB, 1), jnp.int32),
        ],
    )(h_pre, gb1r, gate_w2, gb2r, receptT)

    pos_flat = pos.reshape(KN)
    src_flat = src.reshape(KN)
    be = be2d.reshape(PB)

    mesh = plsc.VectorSubcoreMesh(core_axis_name="c", subcore_axis_name="s")
    dispatch = pl.kernel(
        _dispatch_body,
        mesh=mesh,
        out_type=jax.ShapeDtypeStruct((P, D), jnp.float32),
        scratch_types=[
            pltpu.VMEM((32,), jnp.int32),
            pltpu.VMEM((32,), jnp.int32),
            pltpu.VMEM((32, D), jnp.float32),
            pltpu.SemaphoreType.DMA,
        ],
    )
    x_sorted = dispatch(feat_flat, src_flat, pos_flat)

    y = pl.pallas_call(
        _grouped_ffn_kernel,
        grid_spec=pltpu.PrefetchScalarGridSpec(
            num_scalar_prefetch=1,
            grid=(PB,),
            in_specs=[
                pl.BlockSpec((BP, D), lambda b, be_r: (b, 0)),
                pl.BlockSpec((1, D, H), lambda b, be_r: (be_r[b], 0, 0)),
                pl.BlockSpec((1, 1, H), lambda b, be_r: (be_r[b], 0, 0)),
                pl.BlockSpec((1, H, O), lambda b, be_r: (be_r[b], 0, 0)),
                pl.BlockSpec((1, 1, O), lambda b, be_r: (be_r[b], 0, 0)),
            ],
            out_specs=pl.BlockSpec((BP, O), lambda b, be_r: (b, 0)),
        ),
        out_shape=jax.ShapeDtypeStruct((P, O), jnp.float32),
    )(be, x_sorted, expert_w1, b1r, expert_w2, b2r)

    gather_back = pl.kernel(
        _gather_back_body,
        mesh=mesh,
        out_type=jax.ShapeDtypeStruct((KN, O), jnp.float32),
        scratch_types=[
            pltpu.VMEM((64,), jnp.int32),
            pltpu.VMEM((64, O), jnp.float32),
            pltpu.SemaphoreType.DMA,
        ],
    )
    z = gather_back(y, pos_flat)

    final_out = pl.pallas_call(
        _combine_kernel,
        grid=(NB,),
        in_specs=[
            pl.BlockSpec((BN, TOPK * O), lambda b: (b, 0)),
            pl.BlockSpec((BN, TOPK), lambda b: (b, 0)),
        ],
        out_specs=pl.BlockSpec((BN, O), lambda b: (b, 0)),
        out_shape=jax.ShapeDtypeStruct((N, O), jnp.float32),
    )(z.reshape(N, TOPK * O), wk)

    ranks = jnp.transpose(ranksT)
    totals = totals_row.reshape(E)
    return final_out, ranks, totals


# speedup vs baseline: 2.4268x; 1.1003x over previous
import jax
import jax.numpy as jnp
from jax import lax
from jax.experimental import pallas as pl
from jax.experimental.pallas import tpu as pltpu
from jax.experimental.pallas import tpu_sc as plsc

E, N, D, H, O, TOPK = 8, 2048, 1024, 2048, 1024, 2
KN = N * TOPK
NB = 8
BN = N // NB
BP = 256
P = KN + E * BP
PB = P // BP
NC, NS = 2, 16
NW = NC * NS
APT = KN // NW


def _gate_hidden_kernel(f_ref, g1_ref, h_ref):
    e = pl.program_id(0)
    hb = pl.program_id(1)
    rows = pl.ds(hb * (N // 2), N // 2)
    prod = jnp.dot(f_ref[0], g1_ref[0], preferred_element_type=jnp.float32)

    @pl.when(e == 0)
    def _():
        h_ref[rows, :] = prod

    @pl.when(e > 0)
    def _():
        h_ref[rows, :] = h_ref[rows, :] + prod


def _routing_kernel(hpre_ref, gb1_ref, gw2_ref, gb2_ref, recept_ref,
                    wk_ref, ranks_ref, totals_ref, pos_ref, src_ref, be_ref):
    h = jnp.maximum(hpre_ref[...] + gb1_ref[...], 0.0)
    logits = jnp.dot(h, gw2_ref[...], preferred_element_type=jnp.float32)
    logits = logits + gb2_ref[...]
    m = jnp.max(logits, axis=1, keepdims=True)
    ex = jnp.exp(logits - m)
    probs = ex / jnp.sum(ex, axis=1, keepdims=True)
    scores = probs + recept_ref[...]

    lane = lax.broadcasted_iota(jnp.int32, scores.shape, 1)
    t1 = jnp.max(scores, axis=1, keepdims=True)
    i1 = jnp.min(jnp.where(scores == t1, lane, E), axis=1, keepdims=True)
    masked = jnp.where(lane == i1, -jnp.inf, scores)
    t2 = jnp.max(masked, axis=1, keepdims=True)
    i2 = jnp.min(jnp.where(masked == t2, lane, E), axis=1, keepdims=True)

    denom = t1 + t2
    wk_ref[...] = jnp.concatenate([t1 / denom, t2 / denom], axis=1)
    ranks = jnp.where(lane == i1, 0, jnp.where(lane == i2, 1, 2)).astype(jnp.int32)
    ranks_ref[...] = ranks
    totals_ref[...] = jnp.sum(ranks, axis=0, keepdims=True)

    lane16 = lax.broadcasted_iota(jnp.int32, (N, 2 * E), 1)
    ek = lane16 % E
    kk = lane16 // E
    sel = jnp.where(kk == 0, i1, i2)
    onehot = (sel == ek).astype(jnp.float32)
    tri = (lax.broadcasted_iota(jnp.int32, (N, N), 1)
           < lax.broadcasted_iota(jnp.int32, (N, N), 0)).astype(jnp.float32)
    cum = jnp.dot(tri, onehot, preferred_element_type=jnp.float32)
    tot = jnp.sum(onehot, axis=0, keepdims=True)
    tot0 = tot[:, 0:E]
    counts = tot0 + tot[:, E:2 * E]
    pc = jnp.ceil(counts / BP) * BP
    u8 = (lax.broadcasted_iota(jnp.int32, (E, E), 0)
          < lax.broadcasted_iota(jnp.int32, (E, E), 1)).astype(jnp.float32)
    starts = jnp.dot(jnp.broadcast_to(pc, (E, E)), u8,
                     preferred_element_type=jnp.float32)[0:1, :]
    cum0 = cum[:, 0:E]
    cum1 = cum[:, E:2 * E] + tot0
    oh0 = onehot[:, 0:E]
    oh1 = onehot[:, E:2 * E]
    pos0 = jnp.sum(oh0 * (starts + cum0), axis=1, keepdims=True)
    pos1 = jnp.sum(oh1 * (starts + cum1), axis=1, keepdims=True)
    pos_ref[...] = jnp.concatenate([pos0, pos1], axis=1).astype(jnp.int32)
    tok = lax.broadcasted_iota(jnp.int32, (N, 1), 0)
    src_ref[...] = jnp.concatenate([i1 * N + tok, i2 * N + tok], axis=1)
    bst = (lax.broadcasted_iota(jnp.int32, (PB, E), 0) * BP).astype(jnp.float32)
    startsb = jnp.broadcast_to(starts, (PB, E))
    be_ref[...] = (jnp.sum((startsb <= bst).astype(jnp.int32), axis=1,
                           keepdims=True) - 1)


def _dispatch_body(feat_ref, src_ref, pos_ref, x_ref, sidx_v, pidx_v, rows_v, sem):
    wid = lax.axis_index("s") * NC + lax.axis_index("c")
    base = wid * APT
    for j in range(APT // 32):
        off = base + j * 32
        pltpu.sync_copy(src_ref.at[pl.ds(off, 32)], sidx_v)
        pltpu.sync_copy(pos_ref.at[pl.ds(off, 32)], pidx_v)
        pltpu.async_copy(feat_ref.at[sidx_v], rows_v, sem).wait()
        pltpu.async_copy(rows_v, x_ref.at[pidx_v], sem).wait()


def _combine_body(y_ref, pos_ref, wk_ref, out_ref, pidx_v, wk_v, rows_v, obuf_v, sem):
    wid = lax.axis_index("s") * NC + lax.axis_index("c")
    tpt = N // NW
    tc = 16

    def chunk(cix, carry):
        tok0 = wid * tpt + cix * tc
        a0 = tok0 * 2
        pltpu.sync_copy(pos_ref.at[pl.ds(a0, 2 * tc)], pidx_v)
        pltpu.sync_copy(wk_ref.at[pl.ds(a0, 2 * tc)], wk_v)
        pltpu.async_copy(y_ref.at[pidx_v], rows_v, sem).wait()
        wva = wk_v[pl.ds(0, 16)]
        wvb = wk_v[pl.ds(16, 16)]
        for tt in range(tc):
            wv = wva if tt < 8 else wvb
            w0 = wv[(2 * tt) % 16]
            w1 = wv[(2 * tt + 1) % 16]
            for c in range(O // 16):
                a = rows_v[2 * tt, pl.ds(c * 16, 16)]
                b = rows_v[2 * tt + 1, pl.ds(c * 16, 16)]
                obuf_v[tt, pl.ds(c * 16, 16)] = a * w0 + b * w1
        pltpu.sync_copy(obuf_v, out_ref.at[pl.ds(tok0, tc)])
        return carry

    lax.fori_loop(0, tpt // tc, chunk, 0)


def _grouped_ffn_kernel(be_ref, x_ref, w1_ref, b1_ref, w2_ref, b2_ref, y_ref):
    x = x_ref[...]
    hid = jnp.maximum(jnp.dot(x, w1_ref[0], preferred_element_type=jnp.float32)
                      + b1_ref[0], 0.0)
    y_ref[...] = jnp.dot(hid, w2_ref[0], preferred_element_type=jnp.float32) + b2_ref[0]


def kernel(features_list, receptivity, expert_w1, expert_b1, expert_w2,
           expert_b2, gate_w1, gate_b1, gate_w2, gate_b2):
    g1 = gate_w1.reshape(E, D, H)
    receptT = jnp.transpose(receptivity, (1, 0, 2))[:, :, 0]
    gb1r = gate_b1.reshape(1, H)
    gb2r = gate_b2.reshape(1, E)
    b1r = expert_b1.reshape(E, 1, H)
    b2r = expert_b2.reshape(E, 1, O)
    feat_flat = features_list.reshape(E * N, D)

    h_pre = pl.pallas_call(
        _gate_hidden_kernel,
        grid=(E, 2),
        in_specs=[
            pl.BlockSpec((1, N // 2, D), lambda e, hb: (e, hb, 0)),
            pl.BlockSpec((1, D, H), lambda e, hb: (e, 0, 0)),
        ],
        out_specs=pl.BlockSpec((N, H), lambda e, hb: (0, 0)),
        out_shape=jax.ShapeDtypeStruct((N, H), jnp.float32),
    )(features_list, g1)

    wk, ranksT, totals_row, pos, src, be2d = pl.pallas_call(
        _routing_kernel,
        grid=(1,),
        in_specs=[
            pl.BlockSpec((N, H), lambda i: (0, 0)),
            pl.BlockSpec((1, H), lambda i: (0, 0)),
            pl.BlockSpec((H, E), lambda i: (0, 0)),
            pl.BlockSpec((1, E), lambda i: (0, 0)),
            pl.BlockSpec((N, E), lambda i: (0, 0)),
        ],
        out_specs=[
            pl.BlockSpec((N, TOPK), lambda i: (0, 0)),
            pl.BlockSpec((N, E), lambda i: (0, 0)),
            pl.BlockSpec((1, E), lambda i: (0, 0)),
            pl.BlockSpec((N, TOPK), lambda i: (0, 0)),
            pl.BlockSpec((N, TOPK), lambda i: (0, 0)),
            pl.BlockSpec((PB, 1), lambda i: (0, 0)),
        ],
        out_shape=[
            jax.ShapeDtypeStruct((N, TOPK), jnp.float32),
            jax.ShapeDtypeStruct((N, E), jnp.int32),
            jax.ShapeDtypeStruct((1, E), jnp.int32),
            jax.ShapeDtypeStruct((N, TOPK), jnp.int32),
            jax.ShapeDtypeStruct((N, TOPK), jnp.int32),
            jax.ShapeDtypeStruct((PB, 1), jnp.int32),
        ],
    )(h_pre, gb1r, gate_w2, gb2r, receptT)

    pos_flat = pos.reshape(KN)
    src_flat = src.reshape(KN)
    be = be2d.reshape(PB)

    mesh = plsc.VectorSubcoreMesh(core_axis_name="c", subcore_axis_name="s")
    dispatch = pl.kernel(
        _dispatch_body,
        mesh=mesh,
        out_type=jax.ShapeDtypeStruct((P, D), jnp.float32),
        scratch_types=[
            pltpu.VMEM((32,), jnp.int32),
            pltpu.VMEM((32,), jnp.int32),
            pltpu.VMEM((32, D), jnp.float32),
            pltpu.SemaphoreType.DMA,
        ],
    )
    x_sorted = dispatch(feat_flat, src_flat, pos_flat)

    y = pl.pallas_call(
        _grouped_ffn_kernel,
        grid_spec=pltpu.PrefetchScalarGridSpec(
            num_scalar_prefetch=1,
            grid=(PB,),
            in_specs=[
                pl.BlockSpec((BP, D), lambda b, be_r: (b, 0)),
                pl.BlockSpec((1, D, H), lambda b, be_r: (be_r[b], 0, 0)),
                pl.BlockSpec((1, 1, H), lambda b, be_r: (be_r[b], 0, 0)),
                pl.BlockSpec((1, H, O), lambda b, be_r: (be_r[b], 0, 0)),
                pl.BlockSpec((1, 1, O), lambda b, be_r: (be_r[b], 0, 0)),
            ],
            out_specs=pl.BlockSpec((BP, O), lambda b, be_r: (b, 0)),
        ),
        out_shape=jax.ShapeDtypeStruct((P, O), jnp.float32),
    )(be, x_sorted, expert_w1, b1r, expert_w2, b2r)

    combine = pl.kernel(
        _combine_body,
        mesh=mesh,
        out_type=jax.ShapeDtypeStruct((N, O), jnp.float32),
        scratch_types=[
            pltpu.VMEM((32,), jnp.int32),
            pltpu.VMEM((32,), jnp.float32),
            pltpu.VMEM((32, O), jnp.float32),
            pltpu.VMEM((16, O), jnp.float32),
            pltpu.SemaphoreType.DMA,
        ],
    )
    final_out = combine(y, pos_flat, wk.reshape(KN))

    ranks = jnp.transpose(ranksT)
    totals = totals_row.reshape(E)
    return final_out, ranks, totals


# speedup vs baseline: 2.4886x; 1.0254x over previous
import jax
import jax.numpy as jnp
from jax import lax
from jax.experimental import pallas as pl
from jax.experimental.pallas import tpu as pltpu
from jax.experimental.pallas import tpu_sc as plsc

E, N, D, H, O, TOPK = 8, 2048, 1024, 2048, 1024, 2
KN = N * TOPK
NB = 8
BN = N // NB
BP = 256
P = KN + E * BP
PB = P // BP
NC, NS = 2, 16
NW = NC * NS
APT = KN // NW


def _gate_hidden_kernel(f_ref, g1_ref, h_ref):
    e = pl.program_id(0)
    hb = pl.program_id(1)
    rows = pl.ds(hb * (N // 2), N // 2)
    prod = jnp.dot(f_ref[0], g1_ref[0], preferred_element_type=jnp.float32)

    @pl.when(e == 0)
    def _():
        h_ref[rows, :] = prod

    @pl.when(e > 0)
    def _():
        h_ref[rows, :] = h_ref[rows, :] + prod


def _routing_kernel(hpre_ref, gb1_ref, gw2_ref, gb2_ref, recept_ref,
                    wk_ref, ranks_ref, totals_ref, pos_ref, src_ref, be_ref):
    h = jnp.maximum(hpre_ref[...] + gb1_ref[...], 0.0)
    logits = jnp.dot(h, gw2_ref[...], preferred_element_type=jnp.float32)
    logits = logits + gb2_ref[...]
    m = jnp.max(logits, axis=1, keepdims=True)
    ex = jnp.exp(logits - m)
    probs = ex / jnp.sum(ex, axis=1, keepdims=True)
    scores = probs + recept_ref[...]

    lane = lax.broadcasted_iota(jnp.int32, scores.shape, 1)
    t1 = jnp.max(scores, axis=1, keepdims=True)
    i1 = jnp.min(jnp.where(scores == t1, lane, E), axis=1, keepdims=True)
    masked = jnp.where(lane == i1, -jnp.inf, scores)
    t2 = jnp.max(masked, axis=1, keepdims=True)
    i2 = jnp.min(jnp.where(masked == t2, lane, E), axis=1, keepdims=True)

    denom = t1 + t2
    wk_ref[...] = jnp.concatenate([t1 / denom, t2 / denom], axis=1)
    ranks = jnp.where(lane == i1, 0, jnp.where(lane == i2, 1, 2)).astype(jnp.int32)
    ranks_ref[...] = ranks
    totals_ref[...] = jnp.sum(ranks, axis=0, keepdims=True)

    lane16 = lax.broadcasted_iota(jnp.int32, (N, 2 * E), 1)
    ek = lane16 % E
    kk = lane16 // E
    sel = jnp.where(kk == 0, i1, i2)
    onehot = (sel == ek).astype(jnp.float32)
    tri = (lax.broadcasted_iota(jnp.int32, (N, N), 1)
           < lax.broadcasted_iota(jnp.int32, (N, N), 0)).astype(jnp.float32)
    cum = jnp.dot(tri, onehot, preferred_element_type=jnp.float32)
    tot = jnp.sum(onehot, axis=0, keepdims=True)
    tot0 = tot[:, 0:E]
    counts = tot0 + tot[:, E:2 * E]
    pc = jnp.ceil(counts / BP) * BP
    u8 = (lax.broadcasted_iota(jnp.int32, (E, E), 0)
          < lax.broadcasted_iota(jnp.int32, (E, E), 1)).astype(jnp.float32)
    starts = jnp.dot(jnp.broadcast_to(pc, (E, E)), u8,
                     preferred_element_type=jnp.float32)[0:1, :]
    cum0 = cum[:, 0:E]
    cum1 = cum[:, E:2 * E] + tot0
    oh0 = onehot[:, 0:E]
    oh1 = onehot[:, E:2 * E]
    pos0 = jnp.sum(oh0 * (starts + cum0), axis=1, keepdims=True)
    pos1 = jnp.sum(oh1 * (starts + cum1), axis=1, keepdims=True)
    pos_ref[...] = jnp.concatenate([pos0, pos1], axis=1).astype(jnp.int32)
    tok = lax.broadcasted_iota(jnp.int32, (N, 1), 0)
    src_ref[...] = jnp.concatenate([i1 * N + tok, i2 * N + tok], axis=1)
    bst = (lax.broadcasted_iota(jnp.int32, (PB, E), 0) * BP).astype(jnp.float32)
    startsb = jnp.broadcast_to(starts, (PB, E))
    be_ref[...] = (jnp.sum((startsb <= bst).astype(jnp.int32), axis=1,
                           keepdims=True) - 1)


def _dispatch_body(feat_ref, src_ref, pos_ref, x_ref,
                   sidx0, sidx1, pidx0, pidx1, rows0, rows1, sem0, sem1, osem):
    wid = lax.axis_index("s") * NC + lax.axis_index("c")
    base = wid * APT
    sidx = (sidx0, sidx1)
    pidx = (pidx0, pidx1)
    rows = (rows0, rows1)
    sems = (sem0, sem1)

    def fire(c, par):
        off = base + c * 32
        pltpu.sync_copy(src_ref.at[pl.ds(off, 32)], sidx[par])
        pltpu.sync_copy(pos_ref.at[pl.ds(off, 32)], pidx[par])
        pltpu.async_copy(feat_ref.at[sidx[par]], rows[par], sems[par])

    nch = APT // 32
    fire(0, 0)

    def body(i, carry):
        c0 = 2 * i
        fire(c0 + 1, 1)
        pltpu.make_async_copy(feat_ref.at[sidx0], rows0, sem0).wait()
        pltpu.async_copy(rows0, x_ref.at[pidx0], osem).wait()

        @pl.when(i < nch // 2 - 1)
        def _():
            fire(c0 + 2, 0)

        pltpu.make_async_copy(feat_ref.at[sidx1], rows1, sem1).wait()
        pltpu.async_copy(rows1, x_ref.at[pidx1], osem).wait()
        return carry

    lax.fori_loop(0, nch // 2, body, 0)


def _combine_body(y_ref, pos_ref, wk_ref, out_ref,
                  pidx0, pidx1, wk0, wk1, rows0, rows1, obuf_v, sem0, sem1):
    wid = lax.axis_index("s") * NC + lax.axis_index("c")
    tpt = N // NW
    tc = 8
    pidx = (pidx0, pidx1)
    wkv = (wk0, wk1)
    rows = (rows0, rows1)
    sems = (sem0, sem1)

    def fire(c, par):
        tok0 = wid * tpt + c * tc
        a0 = tok0 * 2
        pltpu.sync_copy(pos_ref.at[pl.ds(a0, 2 * tc)], pidx[par])
        pltpu.sync_copy(wk_ref.at[pl.ds(a0, 2 * tc)], wkv[par])
        pltpu.async_copy(y_ref.at[pidx[par]], rows[par], sems[par])

    def combine_chunk(c, par):
        rv = rows[par]
        wv = wkv[par][...]
        for tt in range(tc):
            w0 = wv[2 * tt]
            w1 = wv[2 * tt + 1]
            for cc in range(O // 16):
                a = rv[2 * tt, pl.ds(cc * 16, 16)]
                b = rv[2 * tt + 1, pl.ds(cc * 16, 16)]
                obuf_v[tt, pl.ds(cc * 16, 16)] = a * w0 + b * w1
        pltpu.sync_copy(obuf_v, out_ref.at[pl.ds(wid * tpt + c * tc, tc)])

    nch = tpt // tc
    fire(0, 0)

    def body(i, carry):
        c0 = 2 * i
        fire(c0 + 1, 1)
        pltpu.make_async_copy(y_ref.at[pidx0], rows0, sem0).wait()
        combine_chunk(c0, 0)

        @pl.when(i < nch // 2 - 1)
        def _():
            fire(c0 + 2, 0)

        pltpu.make_async_copy(y_ref.at[pidx1], rows1, sem1).wait()
        combine_chunk(c0 + 1, 1)
        return carry

    lax.fori_loop(0, nch // 2, body, 0)


def _grouped_ffn_kernel(be_ref, x_ref, w1_ref, b1_ref, w2_ref, b2_ref, y_ref):
    x = x_ref[...]
    hid = jnp.maximum(jnp.dot(x, w1_ref[0], preferred_element_type=jnp.float32)
                      + b1_ref[0], 0.0)
    y_ref[...] = jnp.dot(hid, w2_ref[0], preferred_element_type=jnp.float32) + b2_ref[0]


def kernel(features_list, receptivity, expert_w1, expert_b1, expert_w2,
           expert_b2, gate_w1, gate_b1, gate_w2, gate_b2):
    g1 = gate_w1.reshape(E, D, H)
    receptT = jnp.transpose(receptivity, (1, 0, 2))[:, :, 0]
    gb1r = gate_b1.reshape(1, H)
    gb2r = gate_b2.reshape(1, E)
    b1r = expert_b1.reshape(E, 1, H)
    b2r = expert_b2.reshape(E, 1, O)
    feat_flat = features_list.reshape(E * N, D)

    h_pre = pl.pallas_call(
        _gate_hidden_kernel,
        grid=(E, 2),
        in_specs=[
            pl.BlockSpec((1, N // 2, D), lambda e, hb: (e, hb, 0)),
            pl.BlockSpec((1, D, H), lambda e, hb: (e, 0, 0)),
        ],
        out_specs=pl.BlockSpec((N, H), lambda e, hb: (0, 0)),
        out_shape=jax.ShapeDtypeStruct((N, H), jnp.float32),
    )(features_list, g1)

    wk, ranksT, totals_row, pos, src, be2d = pl.pallas_call(
        _routing_kernel,
        grid=(1,),
        in_specs=[
            pl.BlockSpec((N, H), lambda i: (0, 0)),
            pl.BlockSpec((1, H), lambda i: (0, 0)),
            pl.BlockSpec((H, E), lambda i: (0, 0)),
            pl.BlockSpec((1, E), lambda i: (0, 0)),
            pl.BlockSpec((N, E), lambda i: (0, 0)),
        ],
        out_specs=[
            pl.BlockSpec((N, TOPK), lambda i: (0, 0)),
            pl.BlockSpec((N, E), lambda i: (0, 0)),
            pl.BlockSpec((1, E), lambda i: (0, 0)),
            pl.BlockSpec((N, TOPK), lambda i: (0, 0)),
            pl.BlockSpec((N, TOPK), lambda i: (0, 0)),
            pl.BlockSpec((PB, 1), lambda i: (0, 0)),
        ],
        out_shape=[
            jax.ShapeDtypeStruct((N, TOPK), jnp.float32),
            jax.ShapeDtypeStruct((N, E), jnp.int32),
            jax.ShapeDtypeStruct((1, E), jnp.int32),
            jax.ShapeDtypeStruct((N, TOPK), jnp.int32),
            jax.ShapeDtypeStruct((N, TOPK), jnp.int32),
            jax.ShapeDtypeStruct((PB, 1), jnp.int32),
        ],
    )(h_pre, gb1r, gate_w2, gb2r, receptT)

    pos_flat = pos.reshape(KN)
    src_flat = src.reshape(KN)
    be = be2d.reshape(PB)

    mesh = plsc.VectorSubcoreMesh(core_axis_name="c", subcore_axis_name="s")
    dispatch = pl.kernel(
        _dispatch_body,
        mesh=mesh,
        out_type=jax.ShapeDtypeStruct((P, D), jnp.float32),
        scratch_types=[
            pltpu.VMEM((32,), jnp.int32),
            pltpu.VMEM((32,), jnp.int32),
            pltpu.VMEM((32,), jnp.int32),
            pltpu.VMEM((32,), jnp.int32),
            pltpu.VMEM((32, D), jnp.float32),
            pltpu.VMEM((32, D), jnp.float32),
            pltpu.SemaphoreType.DMA,
            pltpu.SemaphoreType.DMA,
            pltpu.SemaphoreType.DMA,
        ],
    )
    x_sorted = dispatch(feat_flat, src_flat, pos_flat)

    y = pl.pallas_call(
        _grouped_ffn_kernel,
        grid_spec=pltpu.PrefetchScalarGridSpec(
            num_scalar_prefetch=1,
            grid=(PB,),
            in_specs=[
                pl.BlockSpec((BP, D), lambda b, be_r: (b, 0)),
                pl.BlockSpec((1, D, H), lambda b, be_r: (be_r[b], 0, 0)),
                pl.BlockSpec((1, 1, H), lambda b, be_r: (be_r[b], 0, 0)),
                pl.BlockSpec((1, H, O), lambda b, be_r: (be_r[b], 0, 0)),
                pl.BlockSpec((1, 1, O), lambda b, be_r: (be_r[b], 0, 0)),
            ],
            out_specs=pl.BlockSpec((BP, O), lambda b, be_r: (b, 0)),
        ),
        out_shape=jax.ShapeDtypeStruct((P, O), jnp.float32),
    )(be, x_sorted, expert_w1, b1r, expert_w2, b2r)

    combine = pl.kernel(
        _combine_body,
        mesh=mesh,
        out_type=jax.ShapeDtypeStruct((N, O), jnp.float32),
        scratch_types=[
            pltpu.VMEM((16,), jnp.int32),
            pltpu.VMEM((16,), jnp.int32),
            pltpu.VMEM((16,), jnp.float32),
            pltpu.VMEM((16,), jnp.float32),
            pltpu.VMEM((16, O), jnp.float32),
            pltpu.VMEM((16, O), jnp.float32),
            pltpu.VMEM((8, O), jnp.float32),
            pltpu.SemaphoreType.DMA,
            pltpu.SemaphoreType.DMA,
        ],
    )
    final_out = combine(y, pos_flat, wk.reshape(KN))

    ranks = jnp.transpose(ranksT)
    totals = totals_row.reshape(E)
    return final_out, ranks, totals


# speedup vs baseline: 2.5910x; 1.0411x over previous
import jax
import jax.numpy as jnp
from jax import lax
from jax.experimental import pallas as pl
from jax.experimental.pallas import tpu as pltpu
from jax.experimental.pallas import tpu_sc as plsc

E, N, D, H, O, TOPK = 8, 2048, 1024, 2048, 1024, 2
KN = N * TOPK
NB = 8
BN = N // NB
BP = 256
P = KN + E * BP
PB = P // BP
NC, NS = 2, 16
NW = NC * NS
APT = KN // NW


def _gate_routing_kernel(f_ref, g1_ref, gb1_ref, gw2_ref, gb2_ref, recept_ref,
                         wk_ref, ranks_ref, totals_ref, pos_ref, src_ref,
                         be_ref, h_scr):
    e = pl.program_id(0)
    hb = pl.program_id(1)

    @pl.when(e < E)
    def _():
        rows = pl.ds(hb * (N // 2), N // 2)
        prod = jnp.dot(f_ref[0], g1_ref[0], preferred_element_type=jnp.float32)

        @pl.when(e == 0)
        def _():
            h_scr[rows, :] = prod

        @pl.when(e > 0)
        def _():
            h_scr[rows, :] = h_scr[rows, :] + prod

    @pl.when((e == E) & (hb == 0))
    def _():
        _routing_step(h_scr, gb1_ref, gw2_ref, gb2_ref, recept_ref,
                      wk_ref, ranks_ref, totals_ref, pos_ref, src_ref, be_ref)


def _routing_step(hpre_ref, gb1_ref, gw2_ref, gb2_ref, recept_ref,
                  wk_ref, ranks_ref, totals_ref, pos_ref, src_ref, be_ref):
    h = jnp.maximum(hpre_ref[...] + gb1_ref[...], 0.0)
    logits = jnp.dot(h, gw2_ref[...], preferred_element_type=jnp.float32)
    logits = logits + gb2_ref[...]
    m = jnp.max(logits, axis=1, keepdims=True)
    ex = jnp.exp(logits - m)
    probs = ex / jnp.sum(ex, axis=1, keepdims=True)
    scores = probs + recept_ref[...]

    lane = lax.broadcasted_iota(jnp.int32, scores.shape, 1)
    t1 = jnp.max(scores, axis=1, keepdims=True)
    i1 = jnp.min(jnp.where(scores == t1, lane, E), axis=1, keepdims=True)
    masked = jnp.where(lane == i1, -jnp.inf, scores)
    t2 = jnp.max(masked, axis=1, keepdims=True)
    i2 = jnp.min(jnp.where(masked == t2, lane, E), axis=1, keepdims=True)

    denom = t1 + t2
    wk_ref[...] = jnp.concatenate([t1 / denom, t2 / denom], axis=1)
    ranks = jnp.where(lane == i1, 0, jnp.where(lane == i2, 1, 2)).astype(jnp.int32)
    ranks_ref[...] = ranks
    totals_ref[...] = jnp.sum(ranks, axis=0, keepdims=True)

    lane16 = lax.broadcasted_iota(jnp.int32, (N, 2 * E), 1)
    ek = lane16 % E
    kk = lane16 // E
    sel = jnp.where(kk == 0, i1, i2)
    onehot = (sel == ek).astype(jnp.float32)
    ck = 512
    cum = jnp.zeros((N, 2 * E), jnp.float32)
    for c in range(N // ck):
        col = lax.broadcasted_iota(jnp.int32, (N, ck), 1) + c * ck
        row = lax.broadcasted_iota(jnp.int32, (N, ck), 0)
        tri_c = (col < row).astype(jnp.float32)
        cum = cum + jnp.dot(tri_c, onehot[c * ck:(c + 1) * ck, :],
                            preferred_element_type=jnp.float32)
    tot = jnp.sum(onehot, axis=0, keepdims=True)
    tot0 = tot[:, 0:E]
    counts = tot0 + tot[:, E:2 * E]
    pc = jnp.ceil(counts / BP) * BP
    u8 = (lax.broadcasted_iota(jnp.int32, (E, E), 0)
          < lax.broadcasted_iota(jnp.int32, (E, E), 1)).astype(jnp.float32)
    starts = jnp.dot(jnp.broadcast_to(pc, (E, E)), u8,
                     preferred_element_type=jnp.float32)[0:1, :]
    cum0 = cum[:, 0:E]
    cum1 = cum[:, E:2 * E] + tot0
    oh0 = onehot[:, 0:E]
    oh1 = onehot[:, E:2 * E]
    pos0 = jnp.sum(oh0 * (starts + cum0), axis=1, keepdims=True)
    pos1 = jnp.sum(oh1 * (starts + cum1), axis=1, keepdims=True)
    pos_ref[...] = jnp.concatenate([pos0, pos1], axis=1).astype(jnp.int32)
    tok = lax.broadcasted_iota(jnp.int32, (N, 1), 0)
    src_ref[...] = jnp.concatenate([i1 * N + tok, i2 * N + tok], axis=1)
    bst = (lax.broadcasted_iota(jnp.int32, (PB, E), 0) * BP).astype(jnp.float32)
    startsb = jnp.broadcast_to(starts, (PB, E))
    be_ref[...] = (jnp.sum((startsb <= bst).astype(jnp.int32), axis=1,
                           keepdims=True) - 1)


def _dispatch_body(feat_ref, src_ref, pos_ref, x_ref,
                   sidx0, sidx1, pidx0, pidx1, rows0, rows1, sem0, sem1, osem):
    wid = lax.axis_index("s") * NC + lax.axis_index("c")
    base = wid * APT
    sidx = (sidx0, sidx1)
    pidx = (pidx0, pidx1)
    rows = (rows0, rows1)
    sems = (sem0, sem1)

    def fire(c, par):
        off = base + c * 32
        pltpu.sync_copy(src_ref.at[pl.ds(off, 32)], sidx[par])
        pltpu.sync_copy(pos_ref.at[pl.ds(off, 32)], pidx[par])
        pltpu.async_copy(feat_ref.at[sidx[par]], rows[par], sems[par])

    nch = APT // 32
    fire(0, 0)

    def body(i, carry):
        c0 = 2 * i
        fire(c0 + 1, 1)
        pltpu.make_async_copy(feat_ref.at[sidx0], rows0, sem0).wait()
        pltpu.async_copy(rows0, x_ref.at[pidx0], osem).wait()

        @pl.when(i < nch // 2 - 1)
        def _():
            fire(c0 + 2, 0)

        pltpu.make_async_copy(feat_ref.at[sidx1], rows1, sem1).wait()
        pltpu.async_copy(rows1, x_ref.at[pidx1], osem).wait()
        return carry

    lax.fori_loop(0, nch // 2, body, 0)


def _combine_body(y_ref, pos_ref, wk_ref, out_ref,
                  pidx0, pidx1, wk0, wk1, rows0, rows1, obuf_v, sem0, sem1):
    wid = lax.axis_index("s") * NC + lax.axis_index("c")
    tpt = N // NW
    tc = 8
    pidx = (pidx0, pidx1)
    wkv = (wk0, wk1)
    rows = (rows0, rows1)
    sems = (sem0, sem1)

    def fire(c, par):
        tok0 = wid * tpt + c * tc
        a0 = tok0 * 2
        pltpu.sync_copy(pos_ref.at[pl.ds(a0, 2 * tc)], pidx[par])
        pltpu.sync_copy(wk_ref.at[pl.ds(a0, 2 * tc)], wkv[par])
        pltpu.async_copy(y_ref.at[pidx[par]], rows[par], sems[par])

    def combine_chunk(c, par):
        rv = rows[par]
        wv = wkv[par][...]
        for tt in range(tc):
            w0 = wv[2 * tt]
            w1 = wv[2 * tt + 1]
            for cc in range(O // 16):
                a = rv[2 * tt, pl.ds(cc * 16, 16)]
                b = rv[2 * tt + 1, pl.ds(cc * 16, 16)]
                obuf_v[tt, pl.ds(cc * 16, 16)] = a * w0 + b * w1
        pltpu.sync_copy(obuf_v, out_ref.at[pl.ds(wid * tpt + c * tc, tc)])

    nch = tpt // tc
    fire(0, 0)

    def body(i, carry):
        c0 = 2 * i
        fire(c0 + 1, 1)
        pltpu.make_async_copy(y_ref.at[pidx0], rows0, sem0).wait()
        combine_chunk(c0, 0)

        @pl.when(i < nch // 2 - 1)
        def _():
            fire(c0 + 2, 0)

        pltpu.make_async_copy(y_ref.at[pidx1], rows1, sem1).wait()
        combine_chunk(c0 + 1, 1)
        return carry

    lax.fori_loop(0, nch // 2, body, 0)


def _grouped_ffn_kernel(be_ref, x_ref, w1_ref, b1_ref, w2_ref, b2_ref, y_ref):
    x = x_ref[...]
    hid = jnp.maximum(jnp.dot(x, w1_ref[0], preferred_element_type=jnp.float32)
                      + b1_ref[0], 0.0)
    y_ref[...] = jnp.dot(hid, w2_ref[0], preferred_element_type=jnp.float32) + b2_ref[0]


def kernel(features_list, receptivity, expert_w1, expert_b1, expert_w2,
           expert_b2, gate_w1, gate_b1, gate_w2, gate_b2):
    g1 = gate_w1.reshape(E, D, H)
    receptT = jnp.transpose(receptivity, (1, 0, 2))[:, :, 0]
    gb1r = gate_b1.reshape(1, H)
    gb2r = gate_b2.reshape(1, E)
    b1r = expert_b1.reshape(E, 1, H)
    b2r = expert_b2.reshape(E, 1, O)
    feat_flat = features_list.reshape(E * N, D)

    wk, ranksT, totals_row, pos, src, be2d = pl.pallas_call(
        _gate_routing_kernel,
        grid=(E + 1, 2),
        in_specs=[
            pl.BlockSpec((1, N // 2, D),
                         lambda e, hb: (jnp.minimum(e, E - 1), hb, 0)),
            pl.BlockSpec((1, D, H),
                         lambda e, hb: (jnp.minimum(e, E - 1), 0, 0)),
            pl.BlockSpec((1, H), lambda e, hb: (0, 0)),
            pl.BlockSpec((H, E), lambda e, hb: (0, 0)),
            pl.BlockSpec((1, E), lambda e, hb: (0, 0)),
            pl.BlockSpec((N, E), lambda e, hb: (0, 0)),
        ],
        out_specs=[
            pl.BlockSpec((N, TOPK), lambda e, hb: (0, 0)),
            pl.BlockSpec((N, E), lambda e, hb: (0, 0)),
            pl.BlockSpec((1, E), lambda e, hb: (0, 0)),
            pl.BlockSpec((N, TOPK), lambda e, hb: (0, 0)),
            pl.BlockSpec((N, TOPK), lambda e, hb: (0, 0)),
            pl.BlockSpec((PB, 1), lambda e, hb: (0, 0)),
        ],
        out_shape=[
            jax.ShapeDtypeStruct((N, TOPK), jnp.float32),
            jax.ShapeDtypeStruct((N, E), jnp.int32),
            jax.ShapeDtypeStruct((1, E), jnp.int32),
            jax.ShapeDtypeStruct((N, TOPK), jnp.int32),
            jax.ShapeDtypeStruct((N, TOPK), jnp.int32),
            jax.ShapeDtypeStruct((PB, 1), jnp.int32),
        ],
        scratch_shapes=[pltpu.VMEM((N, H), jnp.float32)],
    )(features_list, g1, gb1r, gate_w2, gb2r, receptT)

    pos_flat = pos.reshape(KN)
    src_flat = src.reshape(KN)
    be = be2d.reshape(PB)

    mesh = plsc.VectorSubcoreMesh(core_axis_name="c", subcore_axis_name="s")
    dispatch = pl.kernel(
        _dispatch_body,
        mesh=mesh,
        out_type=jax.ShapeDtypeStruct((P, D), jnp.float32),
        scratch_types=[
            pltpu.VMEM((32,), jnp.int32),
            pltpu.VMEM((32,), jnp.int32),
            pltpu.VMEM((32,), jnp.int32),
            pltpu.VMEM((32,), jnp.int32),
            pltpu.VMEM((32, D), jnp.float32),
            pltpu.VMEM((32, D), jnp.float32),
            pltpu.SemaphoreType.DMA,
            pltpu.SemaphoreType.DMA,
            pltpu.SemaphoreType.DMA,
        ],
    )
    x_sorted = dispatch(feat_flat, src_flat, pos_flat)

    y = pl.pallas_call(
        _grouped_ffn_kernel,
        grid_spec=pltpu.PrefetchScalarGridSpec(
            num_scalar_prefetch=1,
            grid=(PB,),
            in_specs=[
                pl.BlockSpec((BP, D), lambda b, be_r: (b, 0)),
                pl.BlockSpec((1, D, H), lambda b, be_r: (be_r[b], 0, 0)),
                pl.BlockSpec((1, 1, H), lambda b, be_r: (be_r[b], 0, 0)),
                pl.BlockSpec((1, H, O), lambda b, be_r: (be_r[b], 0, 0)),
                pl.BlockSpec((1, 1, O), lambda b, be_r: (be_r[b], 0, 0)),
            ],
            out_specs=pl.BlockSpec((BP, O), lambda b, be_r: (b, 0)),
        ),
        out_shape=jax.ShapeDtypeStruct((P, O), jnp.float32),
    )(be, x_sorted, expert_w1, b1r, expert_w2, b2r)

    combine = pl.kernel(
        _combine_body,
        mesh=mesh,
        out_type=jax.ShapeDtypeStruct((N, O), jnp.float32),
        scratch_types=[
            pltpu.VMEM((16,), jnp.int32),
            pltpu.VMEM((16,), jnp.int32),
            pltpu.VMEM((16,), jnp.float32),
            pltpu.VMEM((16,), jnp.float32),
            pltpu.VMEM((16, O), jnp.float32),
            pltpu.VMEM((16, O), jnp.float32),
            pltpu.VMEM((8, O), jnp.float32),
            pltpu.SemaphoreType.DMA,
            pltpu.SemaphoreType.DMA,
        ],
    )
    final_out = combine(y, pos_flat, wk.reshape(KN))

    ranks = jnp.transpose(ranksT)
    totals = totals_row.reshape(E)
    return final_out, ranks, totals
